# baseline (speedup 1.0000x reference)
import jax
import jax.numpy as jnp
from jax import lax
from jax.experimental import pallas as pl
from jax.experimental.pallas import tpu as pltpu
from jax.experimental.pallas import tpu_sc as plsc

EPS = 1e-05

N_NODES = 10000
D_FEAT = 256
N_PAIRS = 160000

NPAD = 10240
BLK = 1024
NBLK = NPAD // BLK
NTRI = NBLK * (NBLK + 1) // 2

NC, NS, L = 2, 16, 16
NW = NC * NS
P_TILE = N_PAIRS // NW
P_PAD = 5120
CHUNK = 128
NCHUNK = P_PAD // CHUNK
NGRP16 = P_PAD // L
FIRE_W = 8
DIAG_PT = NPAD // NS


def _mm_body(a_ref, b_ref, o_ref):
    o_ref[...] = jnp.dot(a_ref[...], b_ref[...],
                         preferred_element_type=jnp.float32).reshape(BLK * BLK)


def _tri_i(k):
    m = 2 * NBLK + 1
    s = jnp.sqrt((m * m - 8 * k - 4).astype(jnp.float32))
    return ((m - s) / 2).astype(jnp.int32)


def _tri_j(k):
    i = _tri_i(k)
    return k - i * (2 * NBLK + 1 - i) // 2 + i


@jax.jit
def _gram_tc(a_pad, at_pad):
    return pl.pallas_call(
        _mm_body,
        grid=(NTRI,),
        in_specs=[
            pl.BlockSpec((BLK, D_FEAT), lambda k: (_tri_i(k), 0)),
            pl.BlockSpec((D_FEAT, BLK), lambda k: (0, _tri_j(k))),
        ],
        out_specs=pl.BlockSpec((BLK * BLK,), lambda k: (k,)),
        out_shape=jax.ShapeDtypeStruct((NTRI * BLK * BLK,), jnp.float32),
        compiler_params=pltpu.CompilerParams(
            dimension_semantics=("arbitrary",)),
    )(a_pad, at_pad)


def _pairs_body(s_flat, xi, yi, out, norms_sp, xidx_v, yidx_v, fxy, fdg,
                sxy, dvals, norms_v, outbuf, semxy, semd):
    cid = lax.axis_index("c")
    sid = lax.axis_index("s")
    wid = sid * NC + cid
    base = pl.multiple_of(wid * P_TILE, 8)

    zero16 = jnp.zeros((L,), jnp.int32)
    for o in range(P_TILE // L * L, P_PAD, L):
        xidx_v[pl.ds(o, L)] = zero16
        yidx_v[pl.ds(o, L)] = zero16
    pltpu.sync_copy(xi.at[pl.ds(base, P_TILE)], xidx_v.at[pl.ds(0, P_TILE)])
    pltpu.sync_copy(yi.at[pl.ds(base, P_TILE)], yidx_v.at[pl.ds(0, P_TILE)])

    def flat_idx(xv, yv):
        bx = lax.shift_right_logical(xv, 10)
        by = lax.shift_right_logical(yv, 10)
        rx = jnp.bitwise_and(xv, BLK - 1)
        cy = jnp.bitwise_and(yv, BLK - 1)
        swap = bx > by
        bmin = jnp.minimum(bx, by)
        bmax = jnp.maximum(bx, by)
        r = jnp.where(swap, cy, rx)
        c = jnp.where(swap, rx, cy)
        blockid = (lax.shift_right_logical(
            bmin * (2 * NBLK + 1 - bmin), 1) + bmax - bmin)
        return (lax.shift_left(blockid, 20)
                + lax.shift_left(r, 10) + c)

    def build(g, carry):
        o = g * L
        xv = xidx_v[pl.ds(o, L)]
        yv = yidx_v[pl.ds(o, L)]
        fxy[pl.ds(o, L)] = flat_idx(xv, yv)
        return carry

    lax.fori_loop(0, NGRP16, build, 0)

    lane = lax.iota(jnp.int32, L)
    dbase = sid * DIAG_PT

    def build_diag(g, carry):
        nv = lane + (dbase + g * L)
        fdg[pl.ds(g * L, L)] = flat_idx(nv, nv)
        return carry

    lax.fori_loop(0, DIAG_PT // L, build_diag, 0)

    def start_diag(c, carry):
        co = pl.multiple_of(c * CHUNK, 8)
        pltpu.async_copy(s_flat.at[fdg.at[pl.ds(co, CHUNK)]],
                         dvals.at[pl.ds(co, CHUNK)], semd)
        return carry

    lax.fori_loop(0, DIAG_PT // CHUNK, start_diag, 0)

    def start_chunk(c):
        co = pl.multiple_of(c * CHUNK, 8)
        pltpu.async_copy(s_flat.at[fxy.at[pl.ds(co, CHUNK)]],
                         sxy.at[pl.ds(co, CHUNK)], semxy)

    def wait_chunk():
        co = pl.ds(0, CHUNK)
        pltpu.make_async_copy(s_flat.at[fxy.at[co]], sxy.at[co], semxy).wait()

    def fire_body(c, carry):
        start_chunk(c)

        @pl.when(c >= FIRE_W)
        def _():
            wait_chunk()

        return carry

    lax.fori_loop(0, NCHUNK, fire_body, 0)

    def drain_diag(c, carry):
        co = pl.ds(0, CHUNK)
        pltpu.make_async_copy(s_flat.at[fdg.at[co]], dvals.at[co],
                              semd).wait()
        return carry

    lax.fori_loop(0, DIAG_PT // CHUNK, drain_diag, 0)
    pltpu.sync_copy(dvals, norms_sp.at[pl.ds(dbase, DIAG_PT)])
    plsc.subcore_barrier()
    pltpu.sync_copy(norms_sp, norms_v)

    def drain_body(c, carry):
        wait_chunk()
        return carry

    lax.fori_loop(0, FIRE_W, drain_body, 0)

    def epi(g, carry):
        o = g * L
        xv = xidx_v[pl.ds(o, L)]
        yv = yidx_v[pl.ds(o, L)]
        vxx = plsc.load_gather(norms_v, [xv])
        vyy = plsc.load_gather(norms_v, [yv])
        dist = vxx + vyy - 2.0 * sxy[pl.ds(o, L)]
        se = jnp.exp(dist * -0.5)
        outbuf[pl.ds(o, L)] = se * (1.0 - EPS) + EPS * EPS
        return carry

    lax.fori_loop(0, NGRP16, epi, 0)

    pltpu.sync_copy(outbuf.at[pl.ds(0, P_TILE)], out.at[pl.ds(base, P_TILE)])


@jax.jit
def _pairs_sc(s_flat, x_idx, y_idx):
    mesh = plsc.VectorSubcoreMesh(core_axis_name="c", subcore_axis_name="s")
    f = pl.kernel(
        _pairs_body,
        out_type=jax.ShapeDtypeStruct((N_PAIRS,), jnp.float32),
        mesh=mesh,
        scratch_types=[
            pltpu.VMEM_SHARED((NPAD,), jnp.float32),
            pltpu.VMEM((P_PAD,), jnp.int32),
            pltpu.VMEM((P_PAD,), jnp.int32),
            pltpu.VMEM((P_PAD,), jnp.int32),
            pltpu.VMEM((DIAG_PT,), jnp.int32),
            pltpu.VMEM((P_PAD,), jnp.float32),
            pltpu.VMEM((DIAG_PT,), jnp.float32),
            pltpu.VMEM((NPAD,), jnp.float32),
            pltpu.VMEM((P_PAD,), jnp.float32),
            pltpu.SemaphoreType.DMA,
            pltpu.SemaphoreType.DMA,
        ],
        compiler_params=pltpu.CompilerParams(
            use_tc_tiling_on_sc=False, needs_layout_passes=False),
    )
    return f(s_flat, x_idx, y_idx)


def kernel(inputs, x_idx, y_idx):
    assert inputs.shape == (N_NODES, D_FEAT)
    assert x_idx.shape == (N_PAIRS,) and y_idx.shape == (N_PAIRS,)
    a = jnp.pad(inputs.astype(jnp.bfloat16), ((0, NPAD - N_NODES), (0, 0)))
    s = _gram_tc(a, a.T)
    return _pairs_sc(s, x_idx, y_idx)
